# baseline (speedup 1.0000x reference)
import functools

import jax
import jax.numpy as jnp
from jax.experimental import pallas as pl

NUM_LAYERS = 3
K = 2048
D = 256
T = 512


def _rvq_kernel(h_ref, w0_ref, w1_ref, w2_ref,
                q_ref, i0_ref, i1_ref, i2_ref):
    res = h_ref[0]
    q_total = jnp.zeros_like(res)
    idx_refs = (i0_ref, i1_ref, i2_ref)
    w_refs = (w0_ref, w1_ref, w2_ref)
    for layer in range(NUM_LAYERS):
        w = w_refs[layer][...]
        r2 = jnp.sum(res * res, axis=1, keepdims=True)
        w2 = jnp.sum(w * w, axis=1)
        cross = jax.lax.dot_general(
            res, w, (((1,), (1,)), ((), ())),
            precision=jax.lax.Precision.DEFAULT)
        dists = r2 - 2.0 * cross + w2[None, :]
        m = jnp.min(dists, axis=1, keepdims=True)
        iota = jax.lax.broadcasted_iota(jnp.int32, dists.shape, 1)
        idx = jnp.min(jnp.where(dists == m, iota, K), axis=1)
        onehot = (iota == idx[:, None]).astype(jnp.float32)
        quant = jax.lax.dot_general(
            onehot, w, (((1,), (0,)), ((), ())),
            precision=jax.lax.Precision.HIGHEST)
        idx_refs[layer][0, 0, :] = idx
        q_total = q_total + quant
        res = res - quant
    q_ref[0] = q_total


@jax.jit
def kernel(h, W0, W1, W2):
    B = h.shape[0]
    grid = (B,)
    w_spec = pl.BlockSpec((K, D), lambda b: (0, 0))
    out_shapes = (
        jax.ShapeDtypeStruct((B, T, D), jnp.float32),
        jax.ShapeDtypeStruct((B, 1, T), jnp.int32),
        jax.ShapeDtypeStruct((B, 1, T), jnp.int32),
        jax.ShapeDtypeStruct((B, 1, T), jnp.int32),
    )
    idx_spec = pl.BlockSpec((1, 1, T), lambda b: (b, 0, 0))
    q, i0, i1, i2 = pl.pallas_call(
        _rvq_kernel,
        grid=grid,
        in_specs=[
            pl.BlockSpec((1, T, D), lambda b: (b, 0, 0)),
            w_spec, w_spec, w_spec,
        ],
        out_specs=(
            pl.BlockSpec((1, T, D), lambda b: (b, 0, 0)),
            idx_spec, idx_spec, idx_spec,
        ),
        out_shape=out_shapes,
    )(h, W0, W1, W2)
    return (q, i0.reshape(B, T), i1.reshape(B, T), i2.reshape(B, T))


# speedup vs baseline: 1.7945x; 1.7945x over previous
import functools

import jax
import jax.numpy as jnp
from jax import lax
from jax.experimental import pallas as pl
from jax.experimental.pallas import tpu as pltpu
from jax.experimental.pallas import tpu_sc as plsc

NUM_LAYERS = 3
K = 2048
D = 256
T = 512
B = 16
N_TOK = B * T

NC = 2
NS = 16
NW = NC * NS
ROWS_PER_W = N_TOK // NW
CHUNK = 128


def _dist_body(n_prev, h_ref, *rest):
    q_refs = rest[:n_prev]
    w_ref, idx_ref = rest[n_prev], rest[n_prev + 1]
    res = h_ref[0]
    for q_ref in q_refs:
        res = res - q_ref[0]
    w = w_ref[...]
    r2 = jnp.sum(res * res, axis=1, keepdims=True)
    w2 = jnp.sum(w * w, axis=1)
    cross = lax.dot_general(
        res, w, (((1,), (1,)), ((), ())),
        precision=lax.Precision.DEFAULT)
    dists = r2 - 2.0 * cross + w2[None, :]
    m = jnp.min(dists, axis=1, keepdims=True)
    iota = lax.broadcasted_iota(jnp.int32, dists.shape, 1)
    idx_ref[0, 0, :] = jnp.min(jnp.where(dists == m, iota, K), axis=1)


def _dist_call(n_prev, h, qs, W):
    row_spec = pl.BlockSpec((1, T, D), lambda b: (b, 0, 0))
    return pl.pallas_call(
        functools.partial(_dist_body, n_prev),
        grid=(B,),
        in_specs=[row_spec] * (1 + n_prev) + [pl.BlockSpec((K, D), lambda b: (0, 0))],
        out_specs=pl.BlockSpec((1, 1, T), lambda b: (b, 0, 0)),
        out_shape=jax.ShapeDtypeStruct((B, 1, T), jnp.int32),
    )(h, *qs, W)


def _gather_body(table_hbm, idx_hbm, out_hbm, idx_v, rows_v, sem):
    wid = lax.axis_index("s") * NC + lax.axis_index("c")
    pltpu.sync_copy(idx_hbm.at[pl.ds(wid * 2, 2)], idx_v)
    cps = [pltpu.async_copy(table_hbm.at[idx_v.at[j]], rows_v.at[j], sem)
           for j in range(2)]
    for j, cp in enumerate(cps):
        cp.wait()
        pltpu.sync_copy(
            rows_v.at[j], out_hbm.at[pl.ds(wid * ROWS_PER_W + j * CHUNK, CHUNK)])


_sc_gather = pl.kernel(
    _gather_body,
    out_type=jax.ShapeDtypeStruct((N_TOK, D), jnp.float32),
    mesh=plsc.VectorSubcoreMesh(core_axis_name="c", subcore_axis_name="s"),
    scratch_types=[
        pltpu.VMEM((2, CHUNK), jnp.int32),
        pltpu.VMEM((2, CHUNK, D), jnp.float32),
        pltpu.SemaphoreType.DMA,
    ],
)


def _add3_body(a_ref, b_ref, c_ref, o_ref):
    o_ref[...] = (a_ref[...] + b_ref[...]) + c_ref[...]


def _add3(a, b, c):
    row_spec = pl.BlockSpec((1, T, D), lambda i: (i, 0, 0))
    return pl.pallas_call(
        _add3_body,
        grid=(B,),
        in_specs=[row_spec] * 3,
        out_specs=row_spec,
        out_shape=jax.ShapeDtypeStruct((B, T, D), jnp.float32),
    )(a, b, c)


@jax.jit
def kernel(h, W0, W1, W2):
    i0 = _dist_call(0, h, (), W0)
    q0 = _sc_gather(W0, i0.reshape(NW * 2, CHUNK)).reshape(B, T, D)
    i1 = _dist_call(1, h, (q0,), W1)
    q1 = _sc_gather(W1, i1.reshape(NW * 2, CHUNK)).reshape(B, T, D)
    i2 = _dist_call(2, h, (q0, q1), W2)
    q2 = _sc_gather(W2, i2.reshape(NW * 2, CHUNK)).reshape(B, T, D)
    qt = _add3(q0, q1, q2)
    return (qt, i0.reshape(B, T), i1.reshape(B, T), i2.reshape(B, T))


# speedup vs baseline: 2.0579x; 1.1468x over previous
import functools

import jax
import jax.numpy as jnp
from jax import lax
from jax.experimental import pallas as pl
from jax.experimental.pallas import tpu as pltpu
from jax.experimental.pallas import tpu_sc as plsc

NUM_LAYERS = 3
K = 2048
D = 256
T = 512
B = 16

CHUNKS = 2
BC = B // CHUNKS
TOK_C = BC * T

NC = 2
NS = 16
NW = NC * NS
ROWS_PER_W = TOK_C // NW
assert ROWS_PER_W <= 128


def _dist_body(n_prev, h_ref, *rest):
    q_refs = rest[:n_prev]
    w_ref, idx_ref = rest[n_prev], rest[n_prev + 1]
    res = h_ref[0]
    for q_ref in q_refs:
        res = res - q_ref[0]
    w = w_ref[...]
    r2 = jnp.sum(res * res, axis=1, keepdims=True)
    w2 = jnp.sum(w * w, axis=1)
    cross = lax.dot_general(
        res, w, (((1,), (1,)), ((), ())),
        precision=lax.Precision.DEFAULT)
    dists = r2 - 2.0 * cross + w2[None, :]
    m = jnp.min(dists, axis=1, keepdims=True)
    iota = lax.broadcasted_iota(jnp.int32, dists.shape, 1)
    idx_ref[0, 0, :] = jnp.min(jnp.where(dists == m, iota, K), axis=1)


def _dist_call(n_prev, h, qs, W):
    row_spec = pl.BlockSpec((1, T, D), lambda b: (b, 0, 0))
    return pl.pallas_call(
        functools.partial(_dist_body, n_prev),
        grid=(BC,),
        in_specs=[row_spec] * (1 + n_prev) + [pl.BlockSpec((K, D), lambda b: (0, 0))],
        out_specs=pl.BlockSpec((1, 1, T), lambda b: (b, 0, 0)),
        out_shape=jax.ShapeDtypeStruct((BC, 1, T), jnp.int32),
    )(h, *qs, W)


def _gather_body(table_hbm, idx_hbm, out_hbm, idx_v, rows_v, sem):
    wid = lax.axis_index("s") * NC + lax.axis_index("c")
    pltpu.sync_copy(idx_hbm.at[wid], idx_v)
    pltpu.async_copy(table_hbm.at[idx_v], rows_v, sem).wait()
    pltpu.sync_copy(rows_v, out_hbm.at[pl.ds(wid * ROWS_PER_W, ROWS_PER_W)])


_sc_gather = pl.kernel(
    _gather_body,
    out_type=jax.ShapeDtypeStruct((TOK_C, D), jnp.float32),
    mesh=plsc.VectorSubcoreMesh(core_axis_name="c", subcore_axis_name="s"),
    scratch_types=[
        pltpu.VMEM((ROWS_PER_W,), jnp.int32),
        pltpu.VMEM((ROWS_PER_W, D), jnp.float32),
        pltpu.SemaphoreType.DMA,
    ],
)


def _add3_body(a_ref, b_ref, c_ref, o_ref):
    o_ref[...] = (a_ref[...] + b_ref[...]) + c_ref[...]


def _add3(a, b, c):
    row_spec = pl.BlockSpec((1, T, D), lambda i: (i, 0, 0))
    return pl.pallas_call(
        _add3_body,
        grid=(BC,),
        in_specs=[row_spec] * 3,
        out_specs=row_spec,
        out_shape=jax.ShapeDtypeStruct((BC, T, D), jnp.float32),
    )(a, b, c)


@jax.jit
def kernel(h, W0, W1, W2):
    ws = (W0, W1, W2)
    hs = [h[c * BC:(c + 1) * BC] for c in range(CHUNKS)]
    idx = [[None] * CHUNKS for _ in range(NUM_LAYERS)]
    q = [[None] * CHUNKS for _ in range(NUM_LAYERS)]
    for l in range(NUM_LAYERS):
        for c in range(CHUNKS):
            idx[l][c] = _dist_call(l, hs[c], tuple(q[m][c] for m in range(l)),
                                   ws[l])
            q[l][c] = _sc_gather(
                ws[l], idx[l][c].reshape(NW, ROWS_PER_W)).reshape(BC, T, D)
    qt = jnp.concatenate(
        [_add3(q[0][c], q[1][c], q[2][c]) for c in range(CHUNKS)], axis=0)
    outs = [jnp.concatenate([idx[l][c].reshape(BC, T) for c in range(CHUNKS)],
                            axis=0) for l in range(NUM_LAYERS)]
    return (qt, outs[0], outs[1], outs[2])
